# Initial kernel scaffold; baseline (speedup 1.0000x reference)
import jax
import jax.numpy as jnp
from jax.experimental import pallas as pl


def kernel(x):
    raise NotImplementedError("write your pallas kernel here")


# speedup vs baseline: 1.5955x; 1.5955x over previous
import functools

import jax
import jax.numpy as jnp
from jax import lax
from jax.experimental import pallas as pl
from jax.experimental.pallas import tpu as pltpu
from jax.experimental.pallas import tpu_sc as plsc

BATCH = 16384
N_FIELDS = 26
CARD = 64
OUT_W = N_FIELDS * CARD

NUM_WORKERS = 32
ROWS_PER_WORKER = BATCH // NUM_WORKERS
CHUNK_ROWS = 32
CHUNKS = ROWS_PER_WORKER // CHUNK_ROWS
CHUNK_X = CHUNK_ROWS * N_FIELDS
CHUNK_OUT = CHUNK_ROWS * OUT_W
X_PER_WORKER = ROWS_PER_WORKER * N_FIELDS
OUT_PER_WORKER = ROWS_PER_WORKER * OUT_W
LANES = 16
IDX_VECS = CHUNK_X // LANES


def _body(x_hbm, out_hbm, x_vmem, base_vmem, buf, sem):
    c = lax.axis_index("c")
    s = lax.axis_index("s")
    wid = s * 2 + c

    ones = jnp.full((LANES,), 1, jnp.int32)
    zeros = jnp.zeros((LANES,), jnp.int32)

    pltpu.sync_copy(x_hbm.at[pl.ds(wid * X_PER_WORKER, X_PER_WORKER)], x_vmem)

    def _splat(v):
        return jnp.full((LANES,), v, jnp.int32)

    def pre(i, _):
        lin = _splat(i * LANES) + lax.iota(jnp.int32, LANES)
        row = lax.div(lin, _splat(N_FIELDS))
        f = lin - row * _splat(N_FIELDS)
        base_vmem[pl.ds(i * LANES, LANES)] = row * _splat(OUT_W) + f * _splat(CARD)
        return 0

    lax.fori_loop(0, IDX_VECS, pre, 0)

    def z(i, _):
        for u in range(8):
            buf[pl.ds(i * 128 + u * LANES, LANES)] = zeros
        return 0

    lax.fori_loop(0, CHUNK_OUT // 128, z, 0)

    def scatter_pass(coff, value_vec):
        def ib(t, _):
            for u in range(4):
                off = t * 64 + u * LANES
                vals = x_vmem[pl.ds(coff + off, LANES)]
                bse = base_vmem[pl.ds(off, LANES)]
                plsc.store_scatter(buf, [bse + vals], value_vec)
            return 0

        lax.fori_loop(0, IDX_VECS // 4, ib, 0)

    obase = wid * OUT_PER_WORKER

    def chunk_body(k, _):
        coff = k * CHUNK_X
        scatter_pass(coff, ones)
        pltpu.sync_copy(buf, out_hbm.at[pl.ds(obase + k * CHUNK_OUT, CHUNK_OUT)])
        scatter_pass(coff, zeros)
        return 0

    lax.fori_loop(0, CHUNKS, chunk_body, 0)


@jax.jit
def _onehot(x_flat):
    mesh = plsc.VectorSubcoreMesh(core_axis_name="c", subcore_axis_name="s")
    f = functools.partial(
        pl.kernel,
        out_type=jax.ShapeDtypeStruct((BATCH * OUT_W,), jnp.int32),
        scratch_types=[
            pltpu.VMEM((X_PER_WORKER,), jnp.int32),
            pltpu.VMEM((CHUNK_X,), jnp.int32),
            pltpu.VMEM((CHUNK_OUT,), jnp.int32),
            pltpu.SemaphoreType.DMA,
        ],
        mesh=mesh,
        compiler_params=pltpu.CompilerParams(needs_layout_passes=False),
    )(_body)
    return f(x_flat)


def kernel(x):
    out = _onehot(x.reshape(-1))
    return out.reshape(BATCH, OUT_W)


# speedup vs baseline: 1.6537x; 1.0365x over previous
import functools

import jax
import jax.numpy as jnp
from jax import lax
from jax.experimental import pallas as pl
from jax.experimental.pallas import tpu as pltpu
from jax.experimental.pallas import tpu_sc as plsc

BATCH = 16384
N_FIELDS = 26
CARD = 64
OUT_W = N_FIELDS * CARD

NUM_WORKERS = 32
ROWS_PER_WORKER = BATCH // NUM_WORKERS
CHUNK_ROWS = 32
CHUNKS = ROWS_PER_WORKER // CHUNK_ROWS
CHUNK_X = CHUNK_ROWS * N_FIELDS
CHUNK_OUT = CHUNK_ROWS * OUT_W
X_PER_WORKER = ROWS_PER_WORKER * N_FIELDS
OUT_PER_WORKER = ROWS_PER_WORKER * OUT_W
LANES = 16
IDX_VECS = CHUNK_X // LANES


def _body(x_hbm, out_hbm, x_vmem, base_vmem, buf0, buf1, sem0, sem1):
    c = lax.axis_index("c")
    s = lax.axis_index("s")
    wid = s * 2 + c

    ones = jnp.full((LANES,), 1, jnp.int32)
    zeros = jnp.zeros((LANES,), jnp.int32)

    pltpu.sync_copy(x_hbm.at[pl.ds(wid * X_PER_WORKER, X_PER_WORKER)], x_vmem)

    def _splat(v):
        return jnp.full((LANES,), v, jnp.int32)

    def pre(i, _):
        lin = _splat(i * LANES) + lax.iota(jnp.int32, LANES)
        row = lax.div(lin, _splat(N_FIELDS))
        f = lin - row * _splat(N_FIELDS)
        base_vmem[pl.ds(i * LANES, LANES)] = row * _splat(OUT_W) + f * _splat(CARD)
        return 0

    lax.fori_loop(0, IDX_VECS, pre, 0)

    def z(i, _):
        for u in range(8):
            buf0[pl.ds(i * 128 + u * LANES, LANES)] = zeros
        for u in range(8):
            buf1[pl.ds(i * 128 + u * LANES, LANES)] = zeros
        return 0

    lax.fori_loop(0, CHUNK_OUT // 128, z, 0)

    def scatter_pass(buf, k, value_vec):
        coff = k * CHUNK_X

        def ib(t, _):
            for u in range(4):
                off = t * 64 + u * LANES
                vals = x_vmem[pl.ds(coff + off, LANES)]
                bse = base_vmem[pl.ds(off, LANES)]
                plsc.store_scatter(buf, [bse + vals], value_vec)
            return 0

        lax.fori_loop(0, IDX_VECS // 4, ib, 0)

    obase = wid * OUT_PER_WORKER

    def dma(buf, k, sem):
        return pltpu.make_async_copy(
            buf, out_hbm.at[pl.ds(obase + k * CHUNK_OUT, CHUNK_OUT)], sem
        )

    scatter_pass(buf0, 0, ones)
    dma(buf0, 0, sem0).start()
    scatter_pass(buf1, 1, ones)
    dma(buf1, 1, sem1).start()

    def chunk_pair(j, _):
        k0 = 2 * j
        k1 = 2 * j + 1
        dma(buf0, k0 - 2, sem0).wait()
        scatter_pass(buf0, k0 - 2, zeros)
        scatter_pass(buf0, k0, ones)
        dma(buf0, k0, sem0).start()
        dma(buf1, k1 - 2, sem1).wait()
        scatter_pass(buf1, k1 - 2, zeros)
        scatter_pass(buf1, k1, ones)
        dma(buf1, k1, sem1).start()
        return 0

    lax.fori_loop(1, CHUNKS // 2, chunk_pair, 0)
    dma(buf0, CHUNKS - 2, sem0).wait()
    dma(buf1, CHUNKS - 1, sem1).wait()


@jax.jit
def _onehot(x_flat):
    mesh = plsc.VectorSubcoreMesh(core_axis_name="c", subcore_axis_name="s")
    f = functools.partial(
        pl.kernel,
        out_type=jax.ShapeDtypeStruct((BATCH * OUT_W,), jnp.int32),
        scratch_types=[
            pltpu.VMEM((X_PER_WORKER,), jnp.int32),
            pltpu.VMEM((CHUNK_X,), jnp.int32),
            pltpu.VMEM((CHUNK_OUT,), jnp.int32),
            pltpu.VMEM((CHUNK_OUT,), jnp.int32),
            pltpu.SemaphoreType.DMA,
            pltpu.SemaphoreType.DMA,
        ],
        mesh=mesh,
        compiler_params=pltpu.CompilerParams(needs_layout_passes=False),
    )(_body)
    return f(x_flat)


def kernel(x):
    out = _onehot(x.reshape(-1))
    return out.reshape(BATCH, OUT_W)


# speedup vs baseline: 4.1126x; 2.4869x over previous
import functools

import jax
import jax.numpy as jnp
from jax import lax
from jax.experimental import pallas as pl
from jax.experimental.pallas import tpu as pltpu
from jax.experimental.pallas import tpu_sc as plsc

BATCH = 16384
N_FIELDS = 26
CARD = 64
OUT_W = N_FIELDS * CARD

NUM_WORKERS = 32
ROWS_PER_WORKER = BATCH // NUM_WORKERS
CHUNK_ROWS = 32
CHUNKS = ROWS_PER_WORKER // CHUNK_ROWS
CHUNK_X = CHUNK_ROWS * N_FIELDS
X_PER_WORKER = ROWS_PER_WORKER * N_FIELDS
LANES = 16
IDX_VECS = CHUNK_X // LANES


def _body(x_hbm, out_hbm, x_vmem, rowpat_vmem, colbase_vmem, buf0, buf1, sem0, sem1):
    c = lax.axis_index("c")
    s = lax.axis_index("s")
    wid = s * 2 + c

    ones = jnp.full((LANES,), 1, jnp.int32)
    zeros = jnp.zeros((LANES,), jnp.int32)

    pltpu.sync_copy(x_hbm.at[pl.ds(wid * X_PER_WORKER, X_PER_WORKER)], x_vmem)

    def _splat(v):
        return jnp.full((LANES,), v, jnp.int32)

    def pre(i, _):
        lin = _splat(i * LANES) + lax.iota(jnp.int32, LANES)
        row = lax.div(lin, _splat(N_FIELDS))
        f = lin - row * _splat(N_FIELDS)
        rowpat_vmem[pl.ds(i * LANES, LANES)] = row
        colbase_vmem[pl.ds(i * LANES, LANES)] = f * _splat(CARD)
        return 0

    lax.fori_loop(0, IDX_VECS, pre, 0)

    def z(r, _):
        for u in range(OUT_W // LANES):
            buf0[r, pl.ds(u * LANES, LANES)] = zeros
        for u in range(OUT_W // LANES):
            buf1[r, pl.ds(u * LANES, LANES)] = zeros
        return 0

    lax.fori_loop(0, CHUNK_ROWS, z, 0)

    def scatter_pass(buf, k, value_vec):
        coff = k * CHUNK_X

        def ib(t, _):
            for u in range(4):
                off = t * 64 + u * LANES
                vals = x_vmem[pl.ds(coff + off, LANES)]
                rows = rowpat_vmem[pl.ds(off, LANES)]
                cols = colbase_vmem[pl.ds(off, LANES)] + vals
                plsc.store_scatter(buf, [rows, cols], value_vec)
            return 0

        lax.fori_loop(0, IDX_VECS // 4, ib, 0)

    rbase = wid * ROWS_PER_WORKER

    def dma(buf, k, sem):
        return pltpu.make_async_copy(
            buf, out_hbm.at[pl.ds(rbase + k * CHUNK_ROWS, CHUNK_ROWS)], sem
        )

    scatter_pass(buf0, 0, ones)
    dma(buf0, 0, sem0).start()
    scatter_pass(buf1, 1, ones)
    dma(buf1, 1, sem1).start()

    def chunk_pair(j, _):
        k0 = 2 * j
        k1 = 2 * j + 1
        dma(buf0, k0 - 2, sem0).wait()
        scatter_pass(buf0, k0 - 2, zeros)
        scatter_pass(buf0, k0, ones)
        dma(buf0, k0, sem0).start()
        dma(buf1, k1 - 2, sem1).wait()
        scatter_pass(buf1, k1 - 2, zeros)
        scatter_pass(buf1, k1, ones)
        dma(buf1, k1, sem1).start()
        return 0

    lax.fori_loop(1, CHUNKS // 2, chunk_pair, 0)
    dma(buf0, CHUNKS - 2, sem0).wait()
    dma(buf1, CHUNKS - 1, sem1).wait()


@jax.jit
def _onehot(x_flat):
    mesh = plsc.VectorSubcoreMesh(core_axis_name="c", subcore_axis_name="s")
    f = functools.partial(
        pl.kernel,
        out_type=jax.ShapeDtypeStruct((BATCH, OUT_W), jnp.int32),
        scratch_types=[
            pltpu.VMEM((X_PER_WORKER,), jnp.int32),
            pltpu.VMEM((CHUNK_X,), jnp.int32),
            pltpu.VMEM((CHUNK_X,), jnp.int32),
            pltpu.VMEM((CHUNK_ROWS, OUT_W), jnp.int32),
            pltpu.VMEM((CHUNK_ROWS, OUT_W), jnp.int32),
            pltpu.SemaphoreType.DMA,
            pltpu.SemaphoreType.DMA,
        ],
        mesh=mesh,
        compiler_params=pltpu.CompilerParams(needs_layout_passes=False),
    )(_body)
    return f(x_flat)


def kernel(x):
    return _onehot(x.reshape(-1))


# speedup vs baseline: 4.3756x; 1.0640x over previous
import functools

import jax
import jax.numpy as jnp
from jax import lax
from jax.experimental import pallas as pl
from jax.experimental.pallas import tpu as pltpu
from jax.experimental.pallas import tpu_sc as plsc

BATCH = 16384
N_FIELDS = 26
CARD = 64
OUT_W = N_FIELDS * CARD

NUM_WORKERS = 32
ROWS_PER_WORKER = BATCH // NUM_WORKERS
CHUNK_ROWS = 16
CHUNKS = ROWS_PER_WORKER // CHUNK_ROWS
CHUNK_X = CHUNK_ROWS * N_FIELDS
X_PER_WORKER = ROWS_PER_WORKER * N_FIELDS
LANES = 16
X_STRIDE = 128


def _body(x_hbm, out_hbm, x_vmem, buf0, buf1, sem0, sem1):
    c = lax.axis_index("c")
    s = lax.axis_index("s")
    wid = s * 2 + c

    ones = jnp.full((LANES,), 1, jnp.int32)
    zeros = jnp.zeros((LANES,), jnp.int32)

    def _splat(v):
        return jnp.full((LANES,), v, jnp.int32)

    pltpu.sync_copy(
        x_hbm.at[pl.ds(wid * ROWS_PER_WORKER * X_STRIDE, ROWS_PER_WORKER * X_STRIDE)],
        x_vmem,
    )

    iota = lax.iota(jnp.int32, LANES)
    col_a = iota * _splat(CARD)
    col_b = (iota + _splat(10)) * _splat(CARD)
    mask_b = iota >= _splat(6)

    def z(r, _):
        for u in range(OUT_W // LANES):
            buf0[r, pl.ds(u * LANES, LANES)] = zeros
        for u in range(OUT_W // LANES):
            buf1[r, pl.ds(u * LANES, LANES)] = zeros
        return 0

    lax.fori_loop(0, CHUNK_ROWS, z, 0)

    def scatter_pass(buf, k, value_vec):
        row0 = k * CHUNK_ROWS

        def ib(r, _):
            rows = _splat(r)
            roff = (row0 + r) * X_STRIDE
            vals_a = x_vmem[pl.ds(roff, LANES)]
            plsc.store_scatter(buf, [rows, col_a + vals_a], value_vec)
            vals_b = x_vmem[pl.ds(roff + N_FIELDS - LANES, LANES)]
            plsc.store_scatter(buf, [rows, col_b + vals_b], value_vec, mask=mask_b)
            return 0

        lax.fori_loop(0, CHUNK_ROWS, ib, 0)

    rbase = wid * ROWS_PER_WORKER

    def dma(buf, k, sem):
        return pltpu.make_async_copy(
            buf, out_hbm.at[pl.ds(rbase + k * CHUNK_ROWS, CHUNK_ROWS)], sem
        )

    scatter_pass(buf0, 0, ones)
    dma(buf0, 0, sem0).start()
    scatter_pass(buf1, 1, ones)
    dma(buf1, 1, sem1).start()

    def chunk_pair(j, _):
        k0 = 2 * j
        k1 = 2 * j + 1
        dma(buf0, k0 - 2, sem0).wait()
        scatter_pass(buf0, k0 - 2, zeros)
        scatter_pass(buf0, k0, ones)
        dma(buf0, k0, sem0).start()
        dma(buf1, k1 - 2, sem1).wait()
        scatter_pass(buf1, k1 - 2, zeros)
        scatter_pass(buf1, k1, ones)
        dma(buf1, k1, sem1).start()
        return 0

    lax.fori_loop(1, CHUNKS // 2, chunk_pair, 0)
    dma(buf0, CHUNKS - 2, sem0).wait()
    dma(buf1, CHUNKS - 1, sem1).wait()


@jax.jit
def _onehot(x):
    xp = jnp.pad(x, ((0, 0), (0, X_STRIDE - N_FIELDS))).reshape(-1)
    mesh = plsc.VectorSubcoreMesh(core_axis_name="c", subcore_axis_name="s")
    f = functools.partial(
        pl.kernel,
        out_type=jax.ShapeDtypeStruct((BATCH, OUT_W), jnp.int32),
        scratch_types=[
            pltpu.VMEM((ROWS_PER_WORKER * X_STRIDE,), jnp.int32),
            pltpu.VMEM((CHUNK_ROWS, OUT_W), jnp.int32),
            pltpu.VMEM((CHUNK_ROWS, OUT_W), jnp.int32),
            pltpu.SemaphoreType.DMA,
            pltpu.SemaphoreType.DMA,
        ],
        mesh=mesh,
        compiler_params=pltpu.CompilerParams(needs_layout_passes=False),
    )(_body)
    return f(xp)


def kernel(x):
    return _onehot(x)


# speedup vs baseline: 5.0983x; 1.1652x over previous
import functools

import jax
import jax.numpy as jnp
from jax import lax
from jax.experimental import pallas as pl
from jax.experimental.pallas import tpu as pltpu
from jax.experimental.pallas import tpu_sc as plsc

BATCH = 16384
N_FIELDS = 26
N_FIELDS_PAD = 32
CARD = 64
OUT_W = N_FIELDS * CARD

NUM_WORKERS = 32
ROWS_PER_WORKER = BATCH // NUM_WORKERS
CHUNK_ROWS = 32
CHUNKS = ROWS_PER_WORKER // CHUNK_ROWS
LANES = 16


def _body(x_hbm, out_hbm, x_vmem, buf0, buf1, sem0, sem1):
    c = lax.axis_index("c")
    s = lax.axis_index("s")
    wid = s * 2 + c

    ones = jnp.full((LANES,), 1, jnp.int32)
    zeros = jnp.zeros((LANES,), jnp.int32)

    def _splat(v):
        return jnp.full((LANES,), v, jnp.int32)

    pltpu.sync_copy(
        x_hbm.at[pl.ds(0, N_FIELDS_PAD), pl.ds(wid * ROWS_PER_WORKER, ROWS_PER_WORKER)],
        x_vmem,
    )

    rows0 = lax.iota(jnp.int32, LANES)
    rows1 = rows0 + _splat(LANES)

    def scatter_pass(buf, k, value_vec):
        row0 = k * CHUNK_ROWS

        def ib(f, _):
            colbase = _splat(f * CARD)
            vals0 = x_vmem[f, pl.ds(row0, LANES)]
            plsc.store_scatter(buf, [rows0, colbase + vals0], value_vec)
            vals1 = x_vmem[f, pl.ds(row0 + LANES, LANES)]
            plsc.store_scatter(buf, [rows1, colbase + vals1], value_vec)
            return 0

        lax.fori_loop(0, N_FIELDS, ib, 0)

    rbase = wid * ROWS_PER_WORKER

    def dma(buf, k, sem):
        return pltpu.make_async_copy(
            buf, out_hbm.at[pl.ds(rbase + k * CHUNK_ROWS, CHUNK_ROWS)], sem
        )

    def zero_buf(buf):
        def z(r, _):
            for u in range(OUT_W // LANES):
                buf[r, pl.ds(u * LANES, LANES)] = zeros
            return 0

        lax.fori_loop(0, CHUNK_ROWS, z, 0)

    zero_buf(buf0)
    scatter_pass(buf0, 0, ones)
    dma(buf0, 0, sem0).start()
    zero_buf(buf1)
    scatter_pass(buf1, 1, ones)
    dma(buf1, 1, sem1).start()

    def chunk_pair(j, _):
        k0 = 2 * j
        k1 = 2 * j + 1
        dma(buf0, k0 - 2, sem0).wait()
        scatter_pass(buf0, k0 - 2, zeros)
        scatter_pass(buf0, k0, ones)
        dma(buf0, k0, sem0).start()
        dma(buf1, k1 - 2, sem1).wait()
        scatter_pass(buf1, k1 - 2, zeros)
        scatter_pass(buf1, k1, ones)
        dma(buf1, k1, sem1).start()
        return 0

    lax.fori_loop(1, CHUNKS // 2, chunk_pair, 0)
    dma(buf0, CHUNKS - 2, sem0).wait()
    dma(buf1, CHUNKS - 1, sem1).wait()


@jax.jit
def _onehot(x):
    xt = jnp.pad(x.T, ((0, N_FIELDS_PAD - N_FIELDS), (0, 0)))
    mesh = plsc.VectorSubcoreMesh(core_axis_name="c", subcore_axis_name="s")
    f = functools.partial(
        pl.kernel,
        out_type=jax.ShapeDtypeStruct((BATCH, OUT_W), jnp.int32),
        scratch_types=[
            pltpu.VMEM((N_FIELDS_PAD, ROWS_PER_WORKER), jnp.int32),
            pltpu.VMEM((CHUNK_ROWS, OUT_W), jnp.int32),
            pltpu.VMEM((CHUNK_ROWS, OUT_W), jnp.int32),
            pltpu.SemaphoreType.DMA,
            pltpu.SemaphoreType.DMA,
        ],
        mesh=mesh,
        compiler_params=pltpu.CompilerParams(needs_layout_passes=False),
    )(_body)
    return f(xt)


def kernel(x):
    return _onehot(x)


# speedup vs baseline: 5.1314x; 1.0065x over previous
import functools

import jax
import jax.numpy as jnp
from jax import lax
from jax.experimental import pallas as pl
from jax.experimental.pallas import tpu as pltpu
from jax.experimental.pallas import tpu_sc as plsc

BATCH = 16384
N_FIELDS = 26
N_FIELDS_PAD = 32
CARD = 64
OUT_W = N_FIELDS * CARD

NUM_WORKERS = 32
ROWS_PER_WORKER = BATCH // NUM_WORKERS
CHUNK_ROWS = 32
CHUNKS = ROWS_PER_WORKER // CHUNK_ROWS
LANES = 16


def _body(x_hbm, out_hbm, x_vmem, buf0, buf1, sem0, sem1):
    c = lax.axis_index("c")
    s = lax.axis_index("s")
    wid = s * 2 + c

    ones = jnp.full((LANES,), 1, jnp.int32)
    zeros = jnp.zeros((LANES,), jnp.int32)

    def _splat(v):
        return jnp.full((LANES,), v, jnp.int32)

    pltpu.sync_copy(
        x_hbm.at[:, pl.ds(wid * ROWS_PER_WORKER, ROWS_PER_WORKER)],
        x_vmem,
    )

    rows0 = lax.iota(jnp.int32, LANES)
    rows1 = rows0 + _splat(LANES)

    def scatter_pass(buf, k, value_vec):
        row0 = k * CHUNK_ROWS

        def ib(f, _):
            colbase = _splat(f * CARD)
            vals0 = x_vmem[f, pl.ds(row0, LANES)]
            plsc.store_scatter(buf, [rows0, colbase + vals0], value_vec)
            vals1 = x_vmem[f, pl.ds(row0 + LANES, LANES)]
            plsc.store_scatter(buf, [rows1, colbase + vals1], value_vec)
            return 0

        lax.fori_loop(0, N_FIELDS, ib, 0)

    rbase = wid * ROWS_PER_WORKER

    def dma(buf, k, sem):
        return pltpu.make_async_copy(
            buf, out_hbm.at[pl.ds(rbase + k * CHUNK_ROWS, CHUNK_ROWS)], sem
        )

    def zero_buf(buf):
        def z(r, _):
            for u in range(OUT_W // LANES):
                buf[r, pl.ds(u * LANES, LANES)] = zeros
            return 0

        lax.fori_loop(0, CHUNK_ROWS, z, 0)

    zero_buf(buf0)
    scatter_pass(buf0, 0, ones)
    dma(buf0, 0, sem0).start()
    zero_buf(buf1)
    scatter_pass(buf1, 1, ones)
    dma(buf1, 1, sem1).start()

    def chunk_pair(j, _):
        k0 = 2 * j
        k1 = 2 * j + 1
        dma(buf0, k0 - 2, sem0).wait()
        scatter_pass(buf0, k0 - 2, zeros)
        scatter_pass(buf0, k0, ones)
        dma(buf0, k0, sem0).start()
        dma(buf1, k1 - 2, sem1).wait()
        scatter_pass(buf1, k1 - 2, zeros)
        scatter_pass(buf1, k1, ones)
        dma(buf1, k1, sem1).start()
        return 0

    lax.fori_loop(1, CHUNKS // 2, chunk_pair, 0)
    dma(buf0, CHUNKS - 2, sem0).wait()
    dma(buf1, CHUNKS - 1, sem1).wait()


@jax.jit
def _onehot(x):
    xt = x.T
    mesh = plsc.VectorSubcoreMesh(core_axis_name="c", subcore_axis_name="s")
    f = functools.partial(
        pl.kernel,
        out_type=jax.ShapeDtypeStruct((BATCH, OUT_W), jnp.int32),
        scratch_types=[
            pltpu.VMEM((N_FIELDS, ROWS_PER_WORKER), jnp.int32),
            pltpu.VMEM((CHUNK_ROWS, OUT_W), jnp.int32),
            pltpu.VMEM((CHUNK_ROWS, OUT_W), jnp.int32),
            pltpu.SemaphoreType.DMA,
            pltpu.SemaphoreType.DMA,
        ],
        mesh=mesh,
        compiler_params=pltpu.CompilerParams(needs_layout_passes=False),
    )(_body)
    return f(xt)


def kernel(x):
    return _onehot(x)


# speedup vs baseline: 5.2590x; 1.0248x over previous
import functools

import jax
import jax.numpy as jnp
from jax import lax
from jax.experimental import pallas as pl
from jax.experimental.pallas import tpu as pltpu
from jax.experimental.pallas import tpu_sc as plsc

BATCH = 16384
N_FIELDS = 26
N_FIELDS_PAD = 32
CARD = 64
OUT_W = N_FIELDS * CARD

NUM_WORKERS = 32
ROWS_PER_WORKER = BATCH // NUM_WORKERS
CHUNK_ROWS = 32
CHUNKS = ROWS_PER_WORKER // CHUNK_ROWS
LANES = 16


def _body(x_hbm, out_hbm, x_vmem, buf0, buf1, sem0, sem1, semx):
    c = lax.axis_index("c")
    s = lax.axis_index("s")
    wid = s * 2 + c

    ones = jnp.full((LANES,), 1, jnp.int32)
    zeros = jnp.zeros((LANES,), jnp.int32)

    def _splat(v):
        return jnp.full((LANES,), v, jnp.int32)

    xcopy = pltpu.make_async_copy(
        x_hbm.at[:, pl.ds(wid * ROWS_PER_WORKER, ROWS_PER_WORKER)], x_vmem, semx
    )
    xcopy.start()

    rows0 = lax.iota(jnp.int32, LANES)
    rows1 = rows0 + _splat(LANES)

    def scatter_pass(buf, k, value_vec):
        row0 = k * CHUNK_ROWS

        def ib(f, _):
            colbase = _splat(f * CARD)
            vals0 = x_vmem[f, pl.ds(row0, LANES)]
            plsc.store_scatter(buf, [rows0, colbase + vals0], value_vec)
            vals1 = x_vmem[f, pl.ds(row0 + LANES, LANES)]
            plsc.store_scatter(buf, [rows1, colbase + vals1], value_vec)
            return 0

        lax.fori_loop(0, N_FIELDS, ib, 0)

    rbase = wid * ROWS_PER_WORKER

    def dma(buf, k, sem):
        return pltpu.make_async_copy(
            buf, out_hbm.at[pl.ds(rbase + k * CHUNK_ROWS, CHUNK_ROWS)], sem
        )

    def zero_buf(buf):
        def z(r, _):
            for u in range(OUT_W // LANES):
                buf[r, pl.ds(u * LANES, LANES)] = zeros
            return 0

        lax.fori_loop(0, CHUNK_ROWS, z, 0)

    zero_buf(buf0)
    xcopy.wait()
    scatter_pass(buf0, 0, ones)
    dma(buf0, 0, sem0).start()
    zero_buf(buf1)
    scatter_pass(buf1, 1, ones)
    dma(buf1, 1, sem1).start()

    def chunk_pair(j, _):
        k0 = 2 * j
        k1 = 2 * j + 1
        dma(buf0, k0 - 2, sem0).wait()
        scatter_pass(buf0, k0 - 2, zeros)
        scatter_pass(buf0, k0, ones)
        dma(buf0, k0, sem0).start()
        dma(buf1, k1 - 2, sem1).wait()
        scatter_pass(buf1, k1 - 2, zeros)
        scatter_pass(buf1, k1, ones)
        dma(buf1, k1, sem1).start()
        return 0

    lax.fori_loop(1, CHUNKS // 2, chunk_pair, 0)
    dma(buf0, CHUNKS - 2, sem0).wait()
    dma(buf1, CHUNKS - 1, sem1).wait()


@jax.jit
def _onehot(x):
    xt = x.T
    mesh = plsc.VectorSubcoreMesh(core_axis_name="c", subcore_axis_name="s")
    f = functools.partial(
        pl.kernel,
        out_type=jax.ShapeDtypeStruct((BATCH, OUT_W), jnp.int32),
        scratch_types=[
            pltpu.VMEM((N_FIELDS, ROWS_PER_WORKER), jnp.int32),
            pltpu.VMEM((CHUNK_ROWS, OUT_W), jnp.int32),
            pltpu.VMEM((CHUNK_ROWS, OUT_W), jnp.int32),
            pltpu.SemaphoreType.DMA,
            pltpu.SemaphoreType.DMA,
            pltpu.SemaphoreType.DMA,
        ],
        mesh=mesh,
        compiler_params=pltpu.CompilerParams(needs_layout_passes=False),
    )(_body)
    return f(xt)


def kernel(x):
    return _onehot(x)
